# baseline (speedup 1.0000x reference)
import functools

import jax
import jax.numpy as jnp
from jax import lax
from jax.experimental import pallas as pl
from jax.experimental.pallas import tpu as pltpu
from jax.experimental.pallas import tpu_sc as plsc

N = 10000
E = 320000
D = 128
H = 64
O = 2

NC = 2
NS = 16
NW = NC * NS
EPW = E // NW
K = 128
EPADW = 10240
NCHUNK = EPADW // K
NG = 8
LOOK = 4
NTRIP = (NCHUNK - 2 * LOOK) // NG
NPAD = 10240
RPT = NPAD // NS
DPT = NPAD // NS


def _mesh():
    return plsc.VectorSubcoreMesh(
        core_axis_name="c", subcore_axis_name="s", num_cores=NC, num_subcores=NS
    )


@functools.cache
def _make_sc_degree():
    return pl.kernel(
        _sc_degree_body,
        out_type=jax.ShapeDtypeStruct((NC, NPAD), jnp.float32),
        mesh=_mesh(),
        compiler_params=pltpu.CompilerParams(use_tc_tiling_on_sc=False),
        scratch_types=[
            pltpu.VMEM_SHARED((NPAD,), jnp.float32),
            pltpu.VMEM((NCHUNK, K), jnp.int32),
            pltpu.VMEM((K,), jnp.float32),
            pltpu.VMEM((DPT,), jnp.float32),
        ],
    )


def _sc_degree_body(col_hbm, zeros_hbm, deg_hbm, acc, colv, ones, iobuf):
    c = lax.axis_index("c")
    s = lax.axis_index("s")
    wid = c * NS + s

    pltpu.sync_copy(zeros_hbm.at[pl.ds(s * DPT, DPT)], iobuf)
    pltpu.sync_copy(iobuf, acc.at[pl.ds(s * DPT, DPT)])

    for i in range(K // 16):
        ones[pl.ds(i * 16, 16)] = jnp.ones((16,), jnp.float32)
    pltpu.sync_copy(col_hbm.at[wid], colv)
    plsc.subcore_barrier()

    def body(j, carry):
        pltpu.sync_copy(ones, acc.at[colv.at[j]], add=True)
        return carry

    lax.fori_loop(0, NCHUNK, body, 0)
    plsc.subcore_barrier()

    pltpu.sync_copy(acc.at[pl.ds(s * DPT, DPT)], iobuf)
    pltpu.sync_copy(iobuf, deg_hbm.at[c, pl.ds(s * DPT, DPT)])


@functools.cache
def _make_sc_aggregate():
    return pl.kernel(
        _sc_aggregate_body,
        out_type=jax.ShapeDtypeStruct((NC, NPAD, H), jnp.float32),
        mesh=_mesh(),
        compiler_params=pltpu.CompilerParams(use_tc_tiling_on_sc=False),
        scratch_types=[
            pltpu.VMEM_SHARED((NPAD, H), jnp.float32),
            pltpu.VMEM((NCHUNK, K), jnp.int32),
            pltpu.VMEM((NCHUNK, K), jnp.int32),
        ]
        + [pltpu.VMEM((K, H), jnp.float32) for _ in range(NG)]
        + [pltpu.SemaphoreType.DMA for _ in range(2 * NG)],
    )


def _sc_aggregate_body(y_hbm, row_hbm, col_hbm, z_hbm, acc, rowv, colv, *bufsem):
    bufs = bufsem[:NG]
    gsem = bufsem[NG:2 * NG]
    ssem = bufsem[2 * NG:]
    c = lax.axis_index("c")
    s = lax.axis_index("s")
    wid = c * NS + s

    pltpu.sync_copy(y_hbm.at[pl.ds(s * RPT, RPT)], acc.at[pl.ds(s * RPT, RPT)])

    pltpu.sync_copy(row_hbm.at[wid], rowv)
    pltpu.sync_copy(col_hbm.at[wid], colv)
    plsc.subcore_barrier()

    def wait_gather(j, b):
        pltpu.make_async_copy(y_hbm.at[rowv.at[j]], bufs[b], gsem[b]).wait()

    def issue_scatter(j, b):
        pltpu.async_copy(bufs[b], acc.at[colv.at[j]], ssem[b], add=True)

    def wait_scatter(j, b):
        pltpu.make_async_copy(bufs[b], acc.at[colv.at[j]], ssem[b]).wait()

    for j in range(LOOK):
        pltpu.async_copy(y_hbm.at[rowv.at[j]], bufs[j], gsem[j])
    for j in range(LOOK):
        wait_gather(j, j)
        issue_scatter(j, j)
        pltpu.async_copy(y_hbm.at[rowv.at[j + LOOK]], bufs[j + LOOK], gsem[j + LOOK])

    def body(t, carry):
        for i in range(NG):
            j = 2 * LOOK + t * NG + i - LOOK
            b = (LOOK + i) % NG
            wait_gather(j, b)
            issue_scatter(j, b)
            wait_scatter(j - LOOK, i)
            pltpu.async_copy(y_hbm.at[rowv.at[j + LOOK]], bufs[i], gsem[i])
        return carry

    lax.fori_loop(0, NTRIP, body, 0)
    for j in range(NCHUNK - LOOK, NCHUNK):
        wait_gather(j, j % NG)
        issue_scatter(j, j % NG)
    for b in range(NG):
        wait_scatter(NCHUNK - NG + b, b)
    plsc.subcore_barrier()

    pltpu.sync_copy(acc.at[pl.ds(s * RPT, RPT)], z_hbm.at[c, pl.ds(s * RPT, RPT)])


def _tc_scale_body(x_ref, w_ref, degs_ref, dinv_ref, y_ref):
    deg = degs_ref[0, :N] + degs_ref[1, :N] + 1.0
    dinv = lax.rsqrt(deg)[:, None]
    dinv_ref[...] = dinv
    xw = jnp.dot(x_ref[...], w_ref[...], preferred_element_type=jnp.float32)
    y_ref[:N] = dinv * xw
    y_ref[N:] = jnp.zeros((NPAD - N, H), jnp.float32)


def _tc_comb_body(z_ref, y_ref, dinv_ref, b_ref, w_ref, yout_ref):
    dinv = dinv_ref[...]
    zsum = z_ref[0, :N] + z_ref[1, :N] - y_ref[:N]
    h = jnp.maximum(b_ref[...] + dinv * zsum, 0.0)
    yout_ref[:N] = dinv * jnp.dot(h, w_ref[...], preferred_element_type=jnp.float32)
    yout_ref[N:] = jnp.zeros((NPAD - N, H), jnp.float32)


def _tc_final_body(z_ref, y_ref, dinv_ref, b_ref, wm1_ref, bm1_ref, wm2_ref, bm2_ref, out_ref):
    dinv = dinv_ref[...]
    zsum = z_ref[0, :N] + z_ref[1, :N] - y_ref[:N]
    h = jnp.maximum(b_ref[...] + dinv * zsum, 0.0)
    m = jnp.maximum(
        jnp.dot(h, wm1_ref[...], preferred_element_type=jnp.float32) + bm1_ref[...],
        0.0,
    )
    out_ref[...] = jnp.dot(m, wm2_ref[...], preferred_element_type=jnp.float32) + bm2_ref[...]


def kernel(x, edge_index, W1, b1, W2, b2, W3, b3, Wm1, bm1, Wm2, bm2):
    npe = EPADW - EPW
    padv = (
        N
        + (jnp.arange(npe, dtype=jnp.int32)[None, :]
           + 15 * jnp.arange(NW, dtype=jnp.int32)[:, None]) % (NPAD - N)
    )
    row = jnp.concatenate([edge_index[0].reshape(NW, EPW), padv], axis=1)
    col = jnp.concatenate([edge_index[1].reshape(NW, EPW), padv], axis=1)
    row = row.reshape(NW, NCHUNK, K)
    col = col.reshape(NW, NCHUNK, K)
    zeros = jnp.zeros((NPAD,), jnp.float32)

    degs = _make_sc_degree()(col, zeros)

    dinv, y1 = pl.pallas_call(
        _tc_scale_body,
        out_shape=(
            jax.ShapeDtypeStruct((N, 1), jnp.float32),
            jax.ShapeDtypeStruct((NPAD, H), jnp.float32),
        ),
    )(x, W1, degs)

    z1 = _make_sc_aggregate()(y1, row, col)

    y2 = pl.pallas_call(
        _tc_comb_body,
        out_shape=jax.ShapeDtypeStruct((NPAD, H), jnp.float32),
    )(z1, y1, dinv, b1, W2)

    z2 = _make_sc_aggregate()(y2, row, col)

    y3 = pl.pallas_call(
        _tc_comb_body,
        out_shape=jax.ShapeDtypeStruct((NPAD, H), jnp.float32),
    )(z2, y2, dinv, b2, W3)

    z3 = _make_sc_aggregate()(y3, row, col)

    out = pl.pallas_call(
        _tc_final_body,
        out_shape=jax.ShapeDtypeStruct((N, O), jnp.float32),
    )(z3, y3, dinv, b3, Wm1, bm1, Wm2, bm2)

    return out


# speedup vs baseline: 1.0494x; 1.0494x over previous
import functools

import jax
import jax.numpy as jnp
from jax import lax
from jax.experimental import pallas as pl
from jax.experimental.pallas import tpu as pltpu
from jax.experimental.pallas import tpu_sc as plsc

N = 10000
E = 320000
D = 128
H = 64
O = 2

NC = 2
NS = 16
NW = NC * NS
EPW = E // NW
K = 128
EPADW = 10240
NCHUNK = EPADW // K
NBUF = 4
NTRIP = NCHUNK // NBUF
NPAD = 10240
RPT = NPAD // NS
DPT = NPAD // NS


def _mesh():
    return plsc.VectorSubcoreMesh(
        core_axis_name="c", subcore_axis_name="s", num_cores=NC, num_subcores=NS
    )


@functools.cache
def _make_sc_degree():
    return pl.kernel(
        _sc_degree_body,
        out_type=jax.ShapeDtypeStruct((NC * NPAD,), jnp.float32),
        mesh=_mesh(),
        compiler_params=pltpu.CompilerParams(use_tc_tiling_on_sc=False),
        scratch_types=[
            pltpu.VMEM_SHARED((NPAD,), jnp.float32),
            pltpu.VMEM((NCHUNK, K), jnp.int32),
            pltpu.VMEM((K,), jnp.float32),
            pltpu.VMEM((DPT,), jnp.float32),
        ],
    )


def _sc_degree_body(col_hbm, zeros_hbm, deg_hbm, acc, colv, ones, iobuf):
    c = lax.axis_index("c")
    s = lax.axis_index("s")
    wid = c * NS + s

    pltpu.sync_copy(zeros_hbm.at[pl.ds(s * DPT, DPT)], iobuf)
    pltpu.sync_copy(iobuf, acc.at[pl.ds(s * DPT, DPT)])

    for i in range(K // 16):
        ones[pl.ds(i * 16, 16)] = jnp.ones((16,), jnp.float32)
    pltpu.sync_copy(col_hbm.at[wid], colv)
    plsc.subcore_barrier()

    def body(j, carry):
        pltpu.sync_copy(ones, acc.at[colv.at[j]], add=True)
        return carry

    lax.fori_loop(0, NCHUNK, body, 0)
    plsc.subcore_barrier()

    pltpu.sync_copy(acc.at[pl.ds(s * DPT, DPT)], iobuf)
    pltpu.sync_copy(iobuf, deg_hbm.at[pl.ds(c * NPAD + s * DPT, DPT)])


@functools.cache
def _make_sc_aggregate():
    return pl.kernel(
        _sc_aggregate_body,
        out_type=jax.ShapeDtypeStruct((NC, NPAD, H), jnp.float32),
        mesh=_mesh(),
        compiler_params=pltpu.CompilerParams(use_tc_tiling_on_sc=False),
        scratch_types=[
            pltpu.VMEM_SHARED((NPAD, H), jnp.float32),
            pltpu.VMEM((NCHUNK, K), jnp.int32),
            pltpu.VMEM((NCHUNK, K), jnp.int32),
        ]
        + [pltpu.VMEM((K, H), jnp.float32) for _ in range(NBUF)]
        + [pltpu.SemaphoreType.DMA for _ in range(NBUF)],
    )


def _sc_aggregate_body(y_hbm, row_hbm, col_hbm, z_hbm, acc, rowv, colv, *bufsem):
    bufs = bufsem[:NBUF]
    sems = bufsem[NBUF:]
    c = lax.axis_index("c")
    s = lax.axis_index("s")
    wid = c * NS + s

    pltpu.sync_copy(y_hbm.at[pl.ds(s * RPT, RPT)], acc.at[pl.ds(s * RPT, RPT)])

    pltpu.sync_copy(row_hbm.at[wid], rowv)
    pltpu.sync_copy(col_hbm.at[wid], colv)
    plsc.subcore_barrier()

    for b in range(NBUF):
        pltpu.async_copy(y_hbm.at[rowv.at[b]], bufs[b], sems[b])

    def body(i, carry):
        for b in range(NBUF):
            j = i * NBUF + b
            pltpu.make_async_copy(y_hbm.at[rowv.at[j]], bufs[b], sems[b]).wait()
            pltpu.sync_copy(bufs[b], acc.at[colv.at[j]], add=True)
            pltpu.async_copy(y_hbm.at[rowv.at[j + NBUF]], bufs[b], sems[b])
        return carry

    lax.fori_loop(0, NTRIP - 1, body, 0)
    for b in range(NBUF):
        j = (NTRIP - 1) * NBUF + b
        pltpu.make_async_copy(y_hbm.at[rowv.at[j]], bufs[b], sems[b]).wait()
        pltpu.sync_copy(bufs[b], acc.at[colv.at[j]], add=True)
    plsc.subcore_barrier()

    pltpu.sync_copy(acc.at[pl.ds(s * RPT, RPT)], z_hbm.at[c, pl.ds(s * RPT, RPT)])


def _tc_scale_body(x_ref, w_ref, degs_ref, dinv_ref, y_ref):
    deg = degs_ref[:NPAD] + degs_ref[NPAD:] + 1.0
    dinv = lax.rsqrt(deg)[:, None]
    dinv_ref[...] = dinv
    xw = jnp.dot(x_ref[...], w_ref[...], preferred_element_type=jnp.float32)
    y_ref[:N] = dinv[:N] * xw
    y_ref[N:] = jnp.zeros((NPAD - N, H), jnp.float32)


def _tc_comb_body(z_ref, y_ref, dinv_ref, b_ref, w_ref, yout_ref):
    dinv = dinv_ref[:N]
    zsum = z_ref[0, :N] + z_ref[1, :N] - y_ref[:N]
    h = jnp.maximum(b_ref[...] + dinv * zsum, 0.0)
    yout_ref[:N] = dinv * jnp.dot(h, w_ref[...], preferred_element_type=jnp.float32)
    yout_ref[N:] = jnp.zeros((NPAD - N, H), jnp.float32)


def _tc_final_body(z_ref, y_ref, dinv_ref, b_ref, wm1_ref, bm1_ref, wm2_ref, bm2_ref, out_ref):
    dinv = dinv_ref[:N]
    zsum = z_ref[0, :N] + z_ref[1, :N] - y_ref[:N]
    h = jnp.maximum(b_ref[...] + dinv * zsum, 0.0)
    m = jnp.maximum(
        jnp.dot(h, wm1_ref[...], preferred_element_type=jnp.float32) + bm1_ref[...],
        0.0,
    )
    out_ref[...] = jnp.dot(m, wm2_ref[...], preferred_element_type=jnp.float32) + bm2_ref[...]


def kernel(x, edge_index, W1, b1, W2, b2, W3, b3, Wm1, bm1, Wm2, bm2):
    npe = EPADW - EPW
    padv = (
        N
        + (jnp.arange(npe, dtype=jnp.int32)[None, :]
           + 15 * jnp.arange(NW, dtype=jnp.int32)[:, None]) % (NPAD - N)
    )
    row = jnp.concatenate([edge_index[0].reshape(NW, EPW), padv], axis=1)
    col = jnp.concatenate([edge_index[1].reshape(NW, EPW), padv], axis=1)
    row = row.reshape(NW, NCHUNK, K)
    col = col.reshape(NW, NCHUNK, K)
    zeros = jnp.zeros((NPAD,), jnp.float32)

    degs = _make_sc_degree()(col, zeros)

    dinv, y1 = pl.pallas_call(
        _tc_scale_body,
        out_shape=(
            jax.ShapeDtypeStruct((NPAD, 1), jnp.float32),
            jax.ShapeDtypeStruct((NPAD, H), jnp.float32),
        ),
    )(x, W1, degs)

    z1 = _make_sc_aggregate()(y1, row, col)

    y2 = pl.pallas_call(
        _tc_comb_body,
        out_shape=jax.ShapeDtypeStruct((NPAD, H), jnp.float32),
    )(z1, y1, dinv, b1, W2)

    z2 = _make_sc_aggregate()(y2, row, col)

    y3 = pl.pallas_call(
        _tc_comb_body,
        out_shape=jax.ShapeDtypeStruct((NPAD, H), jnp.float32),
    )(z2, y2, dinv, b2, W3)

    z3 = _make_sc_aggregate()(y3, row, col)

    out = pl.pallas_call(
        _tc_final_body,
        out_shape=jax.ShapeDtypeStruct((N, O), jnp.float32),
    )(z3, y3, dinv, b3, Wm1, bm1, Wm2, bm2)

    return out


# speedup vs baseline: 1.1297x; 1.0765x over previous
import functools

import jax
import jax.numpy as jnp
from jax import lax
from jax.experimental import pallas as pl
from jax.experimental.pallas import tpu as pltpu
from jax.experimental.pallas import tpu_sc as plsc

N = 10000
E = 320000
D = 128
H = 64
O = 2

NC = 2
NS = 16
NW = NC * NS
EPW = E // NW
K = 128
EPADW = 10240
NCHUNK = EPADW // K
NBUF = 6
NTRIP = NCHUNK // NBUF
NPAD = 10240
RPT = NPAD // NS
DPT = NPAD // NS


def _mesh():
    return plsc.VectorSubcoreMesh(
        core_axis_name="c", subcore_axis_name="s", num_cores=NC, num_subcores=NS
    )


@functools.cache
def _make_sc_degree():
    return pl.kernel(
        _sc_degree_body,
        out_type=jax.ShapeDtypeStruct((NC * NPAD,), jnp.float32),
        mesh=_mesh(),
        compiler_params=pltpu.CompilerParams(use_tc_tiling_on_sc=False),
        scratch_types=[
            pltpu.VMEM_SHARED((NPAD,), jnp.float32),
            pltpu.VMEM((NCHUNK, K), jnp.int32),
            pltpu.VMEM((K,), jnp.float32),
            pltpu.VMEM((DPT,), jnp.float32),
        ],
    )


def _sc_degree_body(edges_hbm, zeros_hbm, deg_hbm, acc, colv, ones, iobuf):
    c = lax.axis_index("c")
    s = lax.axis_index("s")
    wid = c * NS + s

    pltpu.sync_copy(zeros_hbm.at[pl.ds(s * DPT, DPT)], iobuf)
    pltpu.sync_copy(iobuf, acc.at[pl.ds(s * DPT, DPT)])

    for i in range(K // 16):
        ones[pl.ds(i * 16, 16)] = jnp.ones((16,), jnp.float32)
    pltpu.sync_copy(edges_hbm.at[1, wid], colv)
    plsc.subcore_barrier()

    def body(j, carry):
        pltpu.sync_copy(ones, acc.at[colv.at[j]], add=True)
        return carry

    lax.fori_loop(0, NCHUNK, body, 0)
    plsc.subcore_barrier()

    pltpu.sync_copy(acc.at[pl.ds(s * DPT, DPT)], iobuf)
    pltpu.sync_copy(iobuf, deg_hbm.at[pl.ds(c * NPAD + s * DPT, DPT)])


@functools.cache
def _make_sc_aggregate():
    return pl.kernel(
        _sc_aggregate_body,
        out_type=jax.ShapeDtypeStruct((NC, NPAD, H), jnp.float32),
        mesh=_mesh(),
        compiler_params=pltpu.CompilerParams(use_tc_tiling_on_sc=False),
        scratch_types=[
            pltpu.VMEM_SHARED((NPAD, H), jnp.float32),
            pltpu.VMEM((NCHUNK, K), jnp.int32),
            pltpu.VMEM((NCHUNK, K), jnp.int32),
        ]
        + [pltpu.VMEM((K, H), jnp.float32) for _ in range(NBUF)]
        + [pltpu.SemaphoreType.DMA for _ in range(NBUF)],
    )


def _sc_aggregate_body(y_hbm, edges_hbm, z_hbm, acc, rowv, colv, *bufsem):
    bufs = bufsem[:NBUF]
    sems = bufsem[NBUF:]
    c = lax.axis_index("c")
    s = lax.axis_index("s")
    wid = c * NS + s

    pltpu.sync_copy(y_hbm.at[pl.ds(s * RPT, RPT)], acc.at[pl.ds(s * RPT, RPT)])

    pltpu.sync_copy(edges_hbm.at[0, wid], rowv)
    pltpu.sync_copy(edges_hbm.at[1, wid], colv)
    plsc.subcore_barrier()

    for b in range(NBUF):
        pltpu.async_copy(y_hbm.at[rowv.at[b]], bufs[b], sems[b])

    def body(i, carry):
        for b in range(NBUF):
            j = i * NBUF + b
            pltpu.make_async_copy(y_hbm.at[rowv.at[j]], bufs[b], sems[b]).wait()
            pltpu.sync_copy(bufs[b], acc.at[colv.at[j]], add=True)
            pltpu.async_copy(y_hbm.at[rowv.at[j + NBUF]], bufs[b], sems[b])
        return carry

    lax.fori_loop(0, NTRIP - 1, body, 0)
    for b in range(NBUF):
        j = (NTRIP - 1) * NBUF + b
        pltpu.make_async_copy(y_hbm.at[rowv.at[j]], bufs[b], sems[b]).wait()
        pltpu.sync_copy(bufs[b], acc.at[colv.at[j]], add=True)
    plsc.subcore_barrier()

    pltpu.sync_copy(acc.at[pl.ds(s * RPT, RPT)], z_hbm.at[c, pl.ds(s * RPT, RPT)])


def _dinv(degs_ref):
    deg = degs_ref[:NPAD] + degs_ref[NPAD:] + 1.0
    return lax.rsqrt(deg)[:N, None]


def _tc_scale_body(x_ref, w_ref, degs_ref, y_ref):
    dinv = _dinv(degs_ref)
    xw = jnp.dot(x_ref[...], w_ref[...], preferred_element_type=jnp.float32)
    y_ref[:N] = dinv * xw
    y_ref[N:] = jnp.zeros((NPAD - N, H), jnp.float32)


def _tc_comb_body(z_ref, y_ref, degs_ref, b_ref, w_ref, yout_ref):
    dinv = _dinv(degs_ref)
    zsum = z_ref[0, :N] + z_ref[1, :N] - y_ref[:N]
    h = jnp.maximum(b_ref[...] + dinv * zsum, 0.0)
    yout_ref[:N] = dinv * jnp.dot(h, w_ref[...], preferred_element_type=jnp.float32)
    yout_ref[N:] = jnp.zeros((NPAD - N, H), jnp.float32)


def _tc_final_body(z_ref, y_ref, degs_ref, b_ref, wm1_ref, bm1_ref, wm2_ref, bm2_ref, out_ref):
    dinv = _dinv(degs_ref)
    zsum = z_ref[0, :N] + z_ref[1, :N] - y_ref[:N]
    h = jnp.maximum(b_ref[...] + dinv * zsum, 0.0)
    m = jnp.maximum(
        jnp.dot(h, wm1_ref[...], preferred_element_type=jnp.float32) + bm1_ref[...],
        0.0,
    )
    out_ref[...] = jnp.dot(m, wm2_ref[...], preferred_element_type=jnp.float32) + bm2_ref[...]


def kernel(x, edge_index, W1, b1, W2, b2, W3, b3, Wm1, bm1, Wm2, bm2):
    npe = EPADW - EPW
    padv = (
        N
        + (jnp.arange(npe, dtype=jnp.int32)[None, :]
           + 15 * jnp.arange(NW, dtype=jnp.int32)[:, None]) % (NPAD - N)
    )
    edges = jnp.concatenate(
        [edge_index.reshape(2, NW, EPW), jnp.broadcast_to(padv, (2, NW, npe))],
        axis=2,
    ).reshape(2, NW, NCHUNK, K)
    zeros = jnp.zeros((NPAD,), jnp.float32)

    degs = _make_sc_degree()(edges, zeros)

    y1 = pl.pallas_call(
        _tc_scale_body,
        out_shape=jax.ShapeDtypeStruct((NPAD, H), jnp.float32),
    )(x, W1, degs)

    z1 = _make_sc_aggregate()(y1, edges)

    y2 = pl.pallas_call(
        _tc_comb_body,
        out_shape=jax.ShapeDtypeStruct((NPAD, H), jnp.float32),
    )(z1, y1, degs, b1, W2)

    z2 = _make_sc_aggregate()(y2, edges)

    y3 = pl.pallas_call(
        _tc_comb_body,
        out_shape=jax.ShapeDtypeStruct((NPAD, H), jnp.float32),
    )(z2, y2, degs, b2, W3)

    z3 = _make_sc_aggregate()(y3, edges)

    out = pl.pallas_call(
        _tc_final_body,
        out_shape=jax.ShapeDtypeStruct((N, O), jnp.float32),
    )(z3, y3, degs, b3, Wm1, bm1, Wm2, bm2)

    return out
